# baseline (speedup 1.0000x reference)
import functools

import jax
import jax.numpy as jnp
from jax import lax
from jax.experimental import pallas as pl
from jax.experimental.pallas import tpu as pltpu
from jax.experimental.pallas import tpu_sc as plsc

N_NODES = 10000
N_EDGES = 320000
D = 128

NC = 2
NS = 16
NW = NC * NS

E_PER_TILE = 10240
E_PAD = E_PER_TILE * NW
IDX_ROWS = E_PAD // 128
ROWS_PER_TILE = IDX_ROWS // NW

ACC_ROWS = 10240

_mesh = plsc.VectorSubcoreMesh(core_axis_name="c", subcore_axis_name="s")


def _worker_id():
    return lax.axis_index("c") * NS + lax.axis_index("s")


def _extract_row(pk_all, c, dst_ring=None, b=0, src_ring=None):
    mask = jnp.full((16,), 0xFFFF, jnp.int32)
    for k in range(8):
        v = pk_all[c, pl.ds(k * 16, 16)]
        if src_ring is not None:
            src_ring[b, pl.ds(k * 16, 16)] = jnp.bitwise_and(v, mask)
        if dst_ring is not None:
            dst_ring[b, pl.ds(k * 16, 16)] = jnp.right_shift(v, 16)


@functools.partial(
    pl.kernel,
    out_type=jax.ShapeDtypeStruct((NC, ACC_ROWS), jnp.float32),
    mesh=_mesh,
    scratch_types=[
        pltpu.VMEM((ROWS_PER_TILE, 128), jnp.int32),
        pltpu.VMEM((8, 128), jnp.int32),
        pltpu.VMEM((128,), jnp.float32),
        pltpu.VMEM((640,), jnp.float32),
        pltpu.VMEM_SHARED((ACC_ROWS,), jnp.float32),
        pltpu.SemaphoreType.DMA,
    ],
)
def _sc_degree(eidx_hbm, out_hbm, pk_all, ring, ones_v, z_v, acc_sh, sem):
    cid = lax.axis_index("c")
    sid = lax.axis_index("s")
    wid = _worker_id()

    pltpu.sync_copy(eidx_hbm.at[pl.ds(wid * ROWS_PER_TILE, ROWS_PER_TILE)],
                    pk_all)

    one16 = jnp.ones((16,), jnp.float32)
    zero16 = jnp.zeros((16,), jnp.float32)
    for j in range(8):
        ones_v[pl.ds(j * 16, 16)] = one16
    for j in range(40):
        z_v[pl.ds(j * 16, 16)] = zero16

    pltpu.sync_copy(z_v, acc_sh.at[pl.ds(sid * 640, 640)])
    plsc.subcore_barrier()

    def blk(g, carry):
        for b in range(8):
            _extract_row(pk_all, g * 8 + b, dst_ring=ring, b=b)
        for b in range(8):
            pltpu.async_copy(ones_v, acc_sh.at[ring.at[b]], sem, add=True)
        for b in range(8):
            pltpu.make_async_copy(ones_v, acc_sh.at[ring.at[0]], sem).wait()
        return carry

    lax.fori_loop(0, ROWS_PER_TILE // 8, blk, 0)
    plsc.subcore_barrier()
    pltpu.sync_copy(acc_sh.at[pl.ds(sid * 640, 640)],
                    out_hbm.at[cid, pl.ds(sid * 640, 640)])


_NBUF = 2


@functools.partial(
    pl.kernel,
    out_type=jax.ShapeDtypeStruct((NC, ACC_ROWS, D), jnp.float32),
    mesh=_mesh,
    scratch_types=(
        [
            pltpu.VMEM((ROWS_PER_TILE, 128), jnp.int32),
            pltpu.VMEM((_NBUF, 128), jnp.int32),
            pltpu.VMEM((_NBUF, 128), jnp.int32),
        ]
        + [pltpu.VMEM((128, D), jnp.float32)] * _NBUF
        + [
            pltpu.VMEM((16, D), jnp.float32),
            pltpu.VMEM_SHARED((ACC_ROWS, D), jnp.float32),
        ]
        + [pltpu.SemaphoreType.DMA] * (2 * _NBUF)
    ),
)
def _sc_aggregate(h_hbm, eidx_hbm, out_hbm, pk_all, sring, dring, *rest):
    rows = rest[:_NBUF]
    z_v = rest[_NBUF]
    acc_sh = rest[_NBUF + 1]
    gsem = rest[_NBUF + 2:_NBUF + 2 + _NBUF]
    ssem = rest[_NBUF + 2 + _NBUF:]

    cid = lax.axis_index("c")
    sid = lax.axis_index("s")
    wid = _worker_id()

    pltpu.sync_copy(eidx_hbm.at[pl.ds(wid * ROWS_PER_TILE, ROWS_PER_TILE)],
                    pk_all)

    for b in range(_NBUF):
        _extract_row(pk_all, b, dst_ring=dring, b=b, src_ring=sring)
        pltpu.async_copy(h_hbm.at[sring.at[b]], rows[b], gsem[b])

    zero16 = jnp.zeros((16,), jnp.float32)
    for r in range(16):
        for c in range(8):
            z_v[r, pl.ds(c * 16, 16)] = zero16

    def zcp(t, carry):
        pltpu.sync_copy(z_v, acc_sh.at[pl.ds(sid * 640 + t * 16, 16)])
        return carry

    lax.fori_loop(0, 40, zcp, 0)
    plsc.subcore_barrier()

    def blk(g, carry):
        for b in range(_NBUF):
            c = g * _NBUF + b
            pltpu.make_async_copy(h_hbm.at[sring.at[b]], rows[b],
                                  gsem[b]).wait()
            pltpu.async_copy(rows[b], acc_sh.at[dring.at[b]], ssem[b],
                             add=True)

            @pl.when(c + _NBUF < ROWS_PER_TILE)
            def _():
                pltpu.make_async_copy(rows[b], acc_sh.at[dring.at[b]],
                                      ssem[b]).wait()
                _extract_row(pk_all, c + _NBUF, dst_ring=dring, b=b,
                             src_ring=sring)
                pltpu.async_copy(h_hbm.at[sring.at[b]], rows[b], gsem[b])
        return carry

    lax.fori_loop(0, ROWS_PER_TILE // _NBUF, blk, 0)
    for b in range(_NBUF):
        pltpu.make_async_copy(rows[b], acc_sh.at[dring.at[b]],
                              ssem[b]).wait()
    plsc.subcore_barrier()
    pltpu.sync_copy(acc_sh.at[pl.ds(sid * 640, 640)],
                    out_hbm.at[cid, pl.ds(sid * 640, 640)])


_GRID = 10
_BR = N_NODES // _GRID


def _dis_of(degp_ref):
    deg = degp_ref[:, 0] + degp_ref[:, 1] + 1.0
    return lax.rsqrt(deg)


def _tc1_body(x_ref, w_ref, degp_ref, o_ref):
    dis = _dis_of(degp_ref)
    h = jnp.dot(x_ref[...], w_ref[...], preferred_element_type=jnp.float32)
    o_ref[...] = h * dis[:, None]


def _tc2_body(agg_ref, hp_ref, degp_ref, b_ref, w_ref, o_ref):
    dis = _dis_of(degp_ref)
    t = (agg_ref[0] + agg_ref[1] + hp_ref[...]) * dis[:, None] + b_ref[...]
    t = jnp.maximum(t, 0.0)
    h = jnp.dot(t, w_ref[...], preferred_element_type=jnp.float32)
    o_ref[...] = h * dis[:, None]


def _tc3_body(agg_ref, hp_ref, degp_ref, b_ref, o_ref):
    dis = _dis_of(degp_ref)
    o_ref[...] = ((agg_ref[0] + agg_ref[1] + hp_ref[...]) * dis[:, None]
                  + b_ref[...])


_ROWS_SPEC = pl.BlockSpec((_BR, D), lambda i: (i, 0))
_W_SPEC = pl.BlockSpec((D, D), lambda i: (0, 0))
_DEG_SPEC = pl.BlockSpec((_BR, NC), lambda i: (i, 0))
_AGG_SPEC = pl.BlockSpec((NC, _BR, D), lambda i: (0, i, 0))
_B_SPEC = pl.BlockSpec((1, D), lambda i: (0, 0))

_tc1 = pl.pallas_call(
    _tc1_body,
    grid=(_GRID,),
    in_specs=[_ROWS_SPEC, _W_SPEC, _DEG_SPEC],
    out_specs=_ROWS_SPEC,
    out_shape=jax.ShapeDtypeStruct((N_NODES, D), jnp.float32),
)

_tc2 = pl.pallas_call(
    _tc2_body,
    grid=(_GRID,),
    in_specs=[_AGG_SPEC, _ROWS_SPEC, _DEG_SPEC, _B_SPEC, _W_SPEC],
    out_specs=_ROWS_SPEC,
    out_shape=jax.ShapeDtypeStruct((N_NODES, D), jnp.float32),
)

_tc3 = pl.pallas_call(
    _tc3_body,
    grid=(_GRID,),
    in_specs=[_AGG_SPEC, _ROWS_SPEC, _DEG_SPEC, _B_SPEC],
    out_specs=_ROWS_SPEC,
    out_shape=jax.ShapeDtypeStruct((N_NODES, D), jnp.float32),
)


def kernel(x, edge_index, W1, b1, W2, b2):
    src = edge_index[0].astype(jnp.int32)
    dst = edge_index[1].astype(jnp.int32)

    npad = E_PAD - N_EDGES
    pad_i = jnp.arange(npad, dtype=jnp.int32)
    pad_pk = (pad_i % N_NODES) | (
        (N_NODES + pad_i % (ACC_ROWS - N_NODES)) << 16)
    eidx = jnp.concatenate([src | (dst << 16), pad_pk]).reshape(IDX_ROWS, 128)

    b1r = b1.reshape(1, D)
    b2r = b2.reshape(1, D)

    degp = _sc_degree(eidx).T
    h1 = _tc1(x, W1, degp)
    agg1 = _sc_aggregate(h1, eidx)
    h2 = _tc2(agg1, h1, degp, b1r, W2)
    agg2 = _sc_aggregate(h2, eidx)
    return _tc3(agg2, h2, degp, b2r)


# speedup vs baseline: 1.0295x; 1.0295x over previous
import functools

import jax
import jax.numpy as jnp
from jax import lax
from jax.experimental import pallas as pl
from jax.experimental.pallas import tpu as pltpu
from jax.experimental.pallas import tpu_sc as plsc

N_NODES = 10000
N_EDGES = 320000
D = 128

NC = 2
NS = 16
NW = NC * NS

IDX_ROWS = N_EDGES // 128
ROWS_PER_TILE = 80

ACC_ROWS = 10240

_mesh = plsc.VectorSubcoreMesh(core_axis_name="c", subcore_axis_name="s")


def _worker_id():
    return lax.axis_index("c") * NS + lax.axis_index("s")


def _extract_row(pk_all, c, dst_ring=None, b=0, src_ring=None):
    mask = jnp.full((16,), 0xFFFF, jnp.int32)
    for k in range(8):
        v = pk_all[c, pl.ds(k * 16, 16)]
        if src_ring is not None:
            src_ring[b, pl.ds(k * 16, 16)] = jnp.bitwise_and(v, mask)
        if dst_ring is not None:
            dst_ring[b, pl.ds(k * 16, 16)] = jnp.right_shift(v, 16)


@functools.partial(
    pl.kernel,
    out_type=jax.ShapeDtypeStruct((NC, ACC_ROWS), jnp.float32),
    mesh=_mesh,
    scratch_types=[
        pltpu.VMEM((ROWS_PER_TILE, 128), jnp.int32),
        pltpu.VMEM((8, 128), jnp.int32),
        pltpu.VMEM((128,), jnp.float32),
        pltpu.VMEM((640,), jnp.float32),
        pltpu.VMEM_SHARED((ACC_ROWS,), jnp.float32),
        pltpu.SemaphoreType.DMA,
    ],
)
def _sc_degree(eidx_hbm, out_hbm, pk_all, ring, ones_v, z_v, acc_sh, sem):
    cid = lax.axis_index("c")
    sid = lax.axis_index("s")
    wid = _worker_id()
    last = wid == NW - 1

    @pl.when(last)
    def _():
        pltpu.sync_copy(eidx_hbm.at[pl.ds((NW - 1) * ROWS_PER_TILE, 20)],
                        pk_all.at[pl.ds(0, 20)])

    @pl.when(jnp.logical_not(last))
    def _():
        pltpu.sync_copy(
            eidx_hbm.at[pl.ds(wid * ROWS_PER_TILE, ROWS_PER_TILE)], pk_all)

    one16 = jnp.ones((16,), jnp.float32)
    zero16 = jnp.zeros((16,), jnp.float32)
    for j in range(8):
        ones_v[pl.ds(j * 16, 16)] = one16
    for j in range(40):
        z_v[pl.ds(j * 16, 16)] = zero16

    pltpu.sync_copy(z_v, acc_sh.at[pl.ds(sid * 640, 640)])
    plsc.subcore_barrier()

    ngroups = jnp.where(last, 5, ROWS_PER_TILE // 4)

    def blk(g, carry):
        for b in range(4):
            _extract_row(pk_all, g * 4 + b, dst_ring=ring, b=b)
        for b in range(4):
            pltpu.async_copy(ones_v, acc_sh.at[ring.at[b]], sem, add=True)
        for b in range(4):
            pltpu.make_async_copy(ones_v, acc_sh.at[ring.at[0]], sem).wait()
        return carry

    lax.fori_loop(0, ngroups, blk, 0)
    plsc.subcore_barrier()
    pltpu.sync_copy(acc_sh.at[pl.ds(sid * 640, 640)],
                    out_hbm.at[cid, pl.ds(sid * 640, 640)])


_NBUF = 2


@functools.partial(
    pl.kernel,
    out_type=jax.ShapeDtypeStruct((NC, ACC_ROWS, D), jnp.float32),
    mesh=_mesh,
    scratch_types=(
        [
            pltpu.VMEM((ROWS_PER_TILE, 128), jnp.int32),
            pltpu.VMEM((_NBUF, 128), jnp.int32),
            pltpu.VMEM((_NBUF, 128), jnp.int32),
        ]
        + [pltpu.VMEM((128, D), jnp.float32)] * _NBUF
        + [
            pltpu.VMEM((16, D), jnp.float32),
            pltpu.VMEM_SHARED((ACC_ROWS, D), jnp.float32),
        ]
        + [pltpu.SemaphoreType.DMA] * (2 * _NBUF)
    ),
)
def _sc_aggregate(h_hbm, eidx_hbm, out_hbm, pk_all, sring, dring, *rest):
    rows = rest[:_NBUF]
    z_v = rest[_NBUF]
    acc_sh = rest[_NBUF + 1]
    gsem = rest[_NBUF + 2:_NBUF + 2 + _NBUF]
    ssem = rest[_NBUF + 2 + _NBUF:]

    cid = lax.axis_index("c")
    sid = lax.axis_index("s")
    wid = _worker_id()
    last = wid == NW - 1
    nrows = jnp.where(last, 20, ROWS_PER_TILE)

    @pl.when(last)
    def _():
        pltpu.sync_copy(eidx_hbm.at[pl.ds((NW - 1) * ROWS_PER_TILE, 20)],
                        pk_all.at[pl.ds(0, 20)])

    @pl.when(jnp.logical_not(last))
    def _():
        pltpu.sync_copy(
            eidx_hbm.at[pl.ds(wid * ROWS_PER_TILE, ROWS_PER_TILE)], pk_all)

    for b in range(_NBUF):
        _extract_row(pk_all, b, dst_ring=dring, b=b, src_ring=sring)
        pltpu.async_copy(h_hbm.at[sring.at[b]], rows[b], gsem[b])

    zero16 = jnp.zeros((16,), jnp.float32)
    for r in range(16):
        for c in range(8):
            z_v[r, pl.ds(c * 16, 16)] = zero16

    def zcp(t, carry):
        pltpu.sync_copy(z_v, acc_sh.at[pl.ds(sid * 640 + t * 16, 16)])
        return carry

    lax.fori_loop(0, 40, zcp, 0)
    plsc.subcore_barrier()

    def blk(g, carry):
        for b in range(_NBUF):
            c = g * _NBUF + b
            pltpu.make_async_copy(h_hbm.at[sring.at[b]], rows[b],
                                  gsem[b]).wait()
            pltpu.async_copy(rows[b], acc_sh.at[dring.at[b]], ssem[b],
                             add=True)

            @pl.when(c + _NBUF < nrows)
            def _():
                pltpu.make_async_copy(rows[b], acc_sh.at[dring.at[b]],
                                      ssem[b]).wait()
                _extract_row(pk_all, c + _NBUF, dst_ring=dring, b=b,
                             src_ring=sring)
                pltpu.async_copy(h_hbm.at[sring.at[b]], rows[b], gsem[b])
        return carry

    lax.fori_loop(0, nrows // _NBUF, blk, 0)
    for b in range(_NBUF):
        pltpu.make_async_copy(rows[b], acc_sh.at[dring.at[b]],
                              ssem[b]).wait()
    plsc.subcore_barrier()
    pltpu.sync_copy(acc_sh.at[pl.ds(sid * 640, 640)],
                    out_hbm.at[cid, pl.ds(sid * 640, 640)])


_GRID = 10
_BR = N_NODES // _GRID


def _dis_of(degp_ref):
    deg = degp_ref[:, 0] + degp_ref[:, 1] + 1.0
    return lax.rsqrt(deg)


def _tc1_body(x_ref, w_ref, degp_ref, o_ref):
    dis = _dis_of(degp_ref)
    h = jnp.dot(x_ref[...], w_ref[...], preferred_element_type=jnp.float32)
    o_ref[...] = h * dis[:, None]


def _tc2_body(agg_ref, hp_ref, degp_ref, b_ref, w_ref, o_ref):
    dis = _dis_of(degp_ref)
    t = (agg_ref[0] + agg_ref[1] + hp_ref[...]) * dis[:, None] + b_ref[...]
    t = jnp.maximum(t, 0.0)
    h = jnp.dot(t, w_ref[...], preferred_element_type=jnp.float32)
    o_ref[...] = h * dis[:, None]


def _tc3_body(agg_ref, hp_ref, degp_ref, b_ref, o_ref):
    dis = _dis_of(degp_ref)
    o_ref[...] = ((agg_ref[0] + agg_ref[1] + hp_ref[...]) * dis[:, None]
                  + b_ref[...])


def _pack_body(e_ref, o_ref):
    o_ref[...] = jnp.bitwise_or(e_ref[0], jnp.left_shift(e_ref[1], 16))


_tc_pack = pl.pallas_call(
    _pack_body,
    in_specs=[pl.BlockSpec((2, IDX_ROWS, 128), lambda: (0, 0, 0))],
    out_specs=pl.BlockSpec((IDX_ROWS, 128), lambda: (0, 0)),
    out_shape=jax.ShapeDtypeStruct((IDX_ROWS, 128), jnp.int32),
)


_ROWS_SPEC = pl.BlockSpec((_BR, D), lambda i: (i, 0))
_W_SPEC = pl.BlockSpec((D, D), lambda i: (0, 0))
_DEG_SPEC = pl.BlockSpec((_BR, NC), lambda i: (i, 0))
_AGG_SPEC = pl.BlockSpec((NC, _BR, D), lambda i: (0, i, 0))
_B_SPEC = pl.BlockSpec((1, D), lambda i: (0, 0))

_tc1 = pl.pallas_call(
    _tc1_body,
    grid=(_GRID,),
    in_specs=[_ROWS_SPEC, _W_SPEC, _DEG_SPEC],
    out_specs=_ROWS_SPEC,
    out_shape=jax.ShapeDtypeStruct((N_NODES, D), jnp.float32),
)

_tc2 = pl.pallas_call(
    _tc2_body,
    grid=(_GRID,),
    in_specs=[_AGG_SPEC, _ROWS_SPEC, _DEG_SPEC, _B_SPEC, _W_SPEC],
    out_specs=_ROWS_SPEC,
    out_shape=jax.ShapeDtypeStruct((N_NODES, D), jnp.float32),
)

_tc3 = pl.pallas_call(
    _tc3_body,
    grid=(_GRID,),
    in_specs=[_AGG_SPEC, _ROWS_SPEC, _DEG_SPEC, _B_SPEC],
    out_specs=_ROWS_SPEC,
    out_shape=jax.ShapeDtypeStruct((N_NODES, D), jnp.float32),
)


def kernel(x, edge_index, W1, b1, W2, b2):
    ei = edge_index.astype(jnp.int32).reshape(2, IDX_ROWS, 128)
    eidx = _tc_pack(ei)

    b1r = b1.reshape(1, D)
    b2r = b2.reshape(1, D)

    degp = _sc_degree(eidx).T
    h1 = _tc1(x, W1, degp)
    agg1 = _sc_aggregate(h1, eidx)
    h2 = _tc2(agg1, h1, degp, b1r, W2)
    agg2 = _sc_aggregate(h2, eidx)
    return _tc3(agg2, h2, degp, b2r)


# speedup vs baseline: 1.0509x; 1.0208x over previous
import functools

import jax
import jax.numpy as jnp
from jax import lax
from jax.experimental import pallas as pl
from jax.experimental.pallas import tpu as pltpu
from jax.experimental.pallas import tpu_sc as plsc

N_NODES = 10000
N_EDGES = 320000
D = 128

NC = 2
NS = 16
NW = NC * NS

IDX_ROWS = N_EDGES // 128
ROWS_PER_TILE = 80

ACC_ROWS = 10240

_mesh = plsc.VectorSubcoreMesh(core_axis_name="c", subcore_axis_name="s")


def _worker_id():
    return lax.axis_index("c") * NS + lax.axis_index("s")


def _extract_row(pk_all, c, dst_ring=None, b=0, src_ring=None):
    mask = jnp.full((16,), 0xFFFF, jnp.int32)
    for k in range(8):
        v = pk_all[c, pl.ds(k * 16, 16)]
        if src_ring is not None:
            src_ring[b, pl.ds(k * 16, 16)] = jnp.bitwise_and(v, mask)
        if dst_ring is not None:
            dst_ring[b, pl.ds(k * 16, 16)] = jnp.right_shift(v, 16)


@functools.partial(
    pl.kernel,
    out_type=jax.ShapeDtypeStruct((NC, ACC_ROWS), jnp.float32),
    mesh=_mesh,
    scratch_types=[
        pltpu.VMEM((ROWS_PER_TILE, 128), jnp.int32),
        pltpu.VMEM((128,), jnp.float32),
        pltpu.VMEM((640,), jnp.float32),
        pltpu.VMEM_SHARED((ACC_ROWS,), jnp.float32),
        pltpu.SemaphoreType.DMA,
    ],
)
def _sc_degree(ei_hbm, out_hbm, didx_all, ones_v, z_v, acc_sh, sem):
    cid = lax.axis_index("c")
    sid = lax.axis_index("s")
    wid = _worker_id()
    last = wid == NW - 1

    @pl.when(last)
    def _():
        pltpu.sync_copy(ei_hbm.at[1, pl.ds((NW - 1) * ROWS_PER_TILE, 20)],
                        didx_all.at[pl.ds(0, 20)])

    @pl.when(jnp.logical_not(last))
    def _():
        pltpu.sync_copy(
            ei_hbm.at[1, pl.ds(wid * ROWS_PER_TILE, ROWS_PER_TILE)],
            didx_all)

    one16 = jnp.ones((16,), jnp.float32)
    zero16 = jnp.zeros((16,), jnp.float32)
    for j in range(8):
        ones_v[pl.ds(j * 16, 16)] = one16
    for j in range(40):
        z_v[pl.ds(j * 16, 16)] = zero16

    pltpu.sync_copy(z_v, acc_sh.at[pl.ds(sid * 640, 640)])
    plsc.subcore_barrier()

    ngroups = jnp.where(last, 5, ROWS_PER_TILE // 4)

    def blk(g, carry):
        for b in range(4):
            pltpu.async_copy(ones_v, acc_sh.at[didx_all.at[g * 4 + b]],
                             sem, add=True)
        for b in range(4):
            pltpu.make_async_copy(ones_v, acc_sh.at[didx_all.at[0]],
                                  sem).wait()
        return carry

    lax.fori_loop(0, ngroups, blk, 0)
    plsc.subcore_barrier()
    pltpu.sync_copy(acc_sh.at[pl.ds(sid * 640, 640)],
                    out_hbm.at[cid, pl.ds(sid * 640, 640)])


_NBUF = 2


@functools.partial(
    pl.kernel,
    out_type=jax.ShapeDtypeStruct((NC, ACC_ROWS, D), jnp.float32),
    mesh=_mesh,
    scratch_types=(
        [
            pltpu.VMEM((ROWS_PER_TILE, 128), jnp.int32),
            pltpu.VMEM((_NBUF, 128), jnp.int32),
            pltpu.VMEM((_NBUF, 128), jnp.int32),
        ]
        + [pltpu.VMEM((128, D), jnp.float32)] * _NBUF
        + [
            pltpu.VMEM((16, D), jnp.float32),
            pltpu.VMEM_SHARED((ACC_ROWS, D), jnp.float32),
        ]
        + [pltpu.SemaphoreType.DMA] * (2 * _NBUF)
    ),
)
def _sc_aggregate(h_hbm, eidx_hbm, out_hbm, pk_all, sring, dring, *rest):
    rows = rest[:_NBUF]
    z_v = rest[_NBUF]
    acc_sh = rest[_NBUF + 1]
    gsem = rest[_NBUF + 2:_NBUF + 2 + _NBUF]
    ssem = rest[_NBUF + 2 + _NBUF:]

    cid = lax.axis_index("c")
    sid = lax.axis_index("s")
    wid = _worker_id()
    last = wid == NW - 1
    nrows = jnp.where(last, 20, ROWS_PER_TILE)

    @pl.when(last)
    def _():
        pltpu.sync_copy(eidx_hbm.at[pl.ds((NW - 1) * ROWS_PER_TILE, 20)],
                        pk_all.at[pl.ds(0, 20)])

    @pl.when(jnp.logical_not(last))
    def _():
        pltpu.sync_copy(
            eidx_hbm.at[pl.ds(wid * ROWS_PER_TILE, ROWS_PER_TILE)], pk_all)

    for b in range(_NBUF):
        _extract_row(pk_all, b, dst_ring=dring, b=b, src_ring=sring)
        pltpu.async_copy(h_hbm.at[sring.at[b]], rows[b], gsem[b])

    zero16 = jnp.zeros((16,), jnp.float32)
    for r in range(16):
        for c in range(8):
            z_v[r, pl.ds(c * 16, 16)] = zero16

    def zcp(t, carry):
        pltpu.sync_copy(z_v, acc_sh.at[pl.ds(sid * 640 + t * 16, 16)])
        return carry

    lax.fori_loop(0, 40, zcp, 0)
    plsc.subcore_barrier()

    def blk(g, carry):
        for b in range(_NBUF):
            c = g * _NBUF + b
            pltpu.make_async_copy(h_hbm.at[sring.at[b]], rows[b],
                                  gsem[b]).wait()
            pltpu.async_copy(rows[b], acc_sh.at[dring.at[b]], ssem[b],
                             add=True)

            @pl.when(c + _NBUF < nrows)
            def _():
                pltpu.make_async_copy(rows[b], acc_sh.at[dring.at[b]],
                                      ssem[b]).wait()
                _extract_row(pk_all, c + _NBUF, dst_ring=dring, b=b,
                             src_ring=sring)
                pltpu.async_copy(h_hbm.at[sring.at[b]], rows[b], gsem[b])
        return carry

    lax.fori_loop(0, nrows // _NBUF, blk, 0)
    for b in range(_NBUF):
        pltpu.make_async_copy(rows[b], acc_sh.at[dring.at[b]],
                              ssem[b]).wait()
    plsc.subcore_barrier()
    pltpu.sync_copy(acc_sh.at[pl.ds(sid * 640, 640)],
                    out_hbm.at[cid, pl.ds(sid * 640, 640)])


_GRID = 10
_BR = N_NODES // _GRID


def _dis_of(degp_ref):
    deg = degp_ref[:, 0] + degp_ref[:, 1] + 1.0
    return lax.rsqrt(deg)


def _tc1_body(x_ref, w_ref, degp_ref, o_ref):
    dis = _dis_of(degp_ref)
    h = jnp.dot(x_ref[...], w_ref[...], preferred_element_type=jnp.float32)
    o_ref[...] = h * dis[:, None]


def _tc2_body(agg_ref, hp_ref, degp_ref, b_ref, w_ref, o_ref):
    dis = _dis_of(degp_ref)
    t = (agg_ref[0] + agg_ref[1] + hp_ref[...]) * dis[:, None] + b_ref[...]
    t = jnp.maximum(t, 0.0)
    h = jnp.dot(t, w_ref[...], preferred_element_type=jnp.float32)
    o_ref[...] = h * dis[:, None]


def _tc3_body(agg_ref, hp_ref, degp_ref, b_ref, o_ref):
    dis = _dis_of(degp_ref)
    o_ref[...] = ((agg_ref[0] + agg_ref[1] + hp_ref[...]) * dis[:, None]
                  + b_ref[...])


def _pack_body(e_ref, o_ref):
    o_ref[...] = jnp.bitwise_or(e_ref[0], jnp.left_shift(e_ref[1], 16))


_tc_pack = pl.pallas_call(
    _pack_body,
    in_specs=[pl.BlockSpec((2, IDX_ROWS, 128), lambda: (0, 0, 0))],
    out_specs=pl.BlockSpec((IDX_ROWS, 128), lambda: (0, 0)),
    out_shape=jax.ShapeDtypeStruct((IDX_ROWS, 128), jnp.int32),
)


_ROWS_SPEC = pl.BlockSpec((_BR, D), lambda i: (i, 0))
_W_SPEC = pl.BlockSpec((D, D), lambda i: (0, 0))
_DEG_SPEC = pl.BlockSpec((_BR, NC), lambda i: (i, 0))
_AGG_SPEC = pl.BlockSpec((NC, _BR, D), lambda i: (0, i, 0))
_B_SPEC = pl.BlockSpec((1, D), lambda i: (0, 0))

_tc1 = pl.pallas_call(
    _tc1_body,
    grid=(_GRID,),
    in_specs=[_ROWS_SPEC, _W_SPEC, _DEG_SPEC],
    out_specs=_ROWS_SPEC,
    out_shape=jax.ShapeDtypeStruct((N_NODES, D), jnp.float32),
)

_tc2 = pl.pallas_call(
    _tc2_body,
    grid=(_GRID,),
    in_specs=[_AGG_SPEC, _ROWS_SPEC, _DEG_SPEC, _B_SPEC, _W_SPEC],
    out_specs=_ROWS_SPEC,
    out_shape=jax.ShapeDtypeStruct((N_NODES, D), jnp.float32),
)

_tc3 = pl.pallas_call(
    _tc3_body,
    grid=(_GRID,),
    in_specs=[_AGG_SPEC, _ROWS_SPEC, _DEG_SPEC, _B_SPEC],
    out_specs=_ROWS_SPEC,
    out_shape=jax.ShapeDtypeStruct((N_NODES, D), jnp.float32),
)


def kernel(x, edge_index, W1, b1, W2, b2):
    ei = edge_index.astype(jnp.int32).reshape(2, IDX_ROWS, 128)

    b1r = b1.reshape(1, D)
    b2r = b2.reshape(1, D)

    degp = _sc_degree(ei).T
    eidx = _tc_pack(ei)
    h1 = _tc1(x, W1, degp)
    agg1 = _sc_aggregate(h1, eidx)
    h2 = _tc2(agg1, h1, degp, b1r, W2)
    agg2 = _sc_aggregate(h2, eidx)
    return _tc3(agg2, h2, degp, b2r)


# speedup vs baseline: 1.1614x; 1.1051x over previous
import functools

import jax
import jax.numpy as jnp
from jax import lax
from jax.experimental import pallas as pl
from jax.experimental.pallas import tpu as pltpu
from jax.experimental.pallas import tpu_sc as plsc

N_NODES = 10000
N_EDGES = 320000
D = 128

NC = 2
NS = 16
NW = NC * NS

IDX_ROWS = N_EDGES // 128
ROWS_PER_TILE = 80

ACC_ROWS = 10240

_mesh = plsc.VectorSubcoreMesh(core_axis_name="c", subcore_axis_name="s")


def _worker_id():
    return lax.axis_index("c") * NS + lax.axis_index("s")


def _extract_row(pk_all, c, dst_ring=None, b=0, src_ring=None):
    mask = jnp.full((16,), 0xFFFF, jnp.int32)
    for k in range(8):
        v = pk_all[c, pl.ds(k * 16, 16)]
        if src_ring is not None:
            src_ring[b, pl.ds(k * 16, 16)] = jnp.bitwise_and(v, mask)
        if dst_ring is not None:
            dst_ring[b, pl.ds(k * 16, 16)] = jnp.right_shift(v, 16)


@functools.partial(
    pl.kernel,
    out_type=jax.ShapeDtypeStruct((NC, ACC_ROWS), jnp.float32),
    mesh=_mesh,
    scratch_types=[
        pltpu.VMEM((ROWS_PER_TILE, 128), jnp.int32),
        pltpu.VMEM((128,), jnp.float32),
        pltpu.VMEM((640,), jnp.float32),
        pltpu.VMEM_SHARED((ACC_ROWS,), jnp.float32),
        pltpu.SemaphoreType.DMA,
    ],
)
def _sc_degree(ei_hbm, out_hbm, didx_all, ones_v, z_v, acc_sh, sem):
    cid = lax.axis_index("c")
    sid = lax.axis_index("s")
    wid = _worker_id()
    last = wid == NW - 1

    @pl.when(last)
    def _():
        pltpu.sync_copy(ei_hbm.at[1, pl.ds((NW - 1) * ROWS_PER_TILE, 20)],
                        didx_all.at[pl.ds(0, 20)])

    @pl.when(jnp.logical_not(last))
    def _():
        pltpu.sync_copy(
            ei_hbm.at[1, pl.ds(wid * ROWS_PER_TILE, ROWS_PER_TILE)],
            didx_all)

    one16 = jnp.ones((16,), jnp.float32)
    zero16 = jnp.zeros((16,), jnp.float32)
    for j in range(8):
        ones_v[pl.ds(j * 16, 16)] = one16
    for j in range(40):
        z_v[pl.ds(j * 16, 16)] = zero16

    pltpu.sync_copy(z_v, acc_sh.at[pl.ds(sid * 640, 640)])
    plsc.subcore_barrier()

    ngroups = jnp.where(last, 5, ROWS_PER_TILE // 4)

    def blk(g, carry):
        for b in range(4):
            pltpu.async_copy(ones_v, acc_sh.at[didx_all.at[g * 4 + b]],
                             sem, add=True)
        for b in range(4):
            pltpu.make_async_copy(ones_v, acc_sh.at[didx_all.at[0]],
                                  sem).wait()
        return carry

    lax.fori_loop(0, ngroups, blk, 0)
    plsc.subcore_barrier()
    pltpu.sync_copy(acc_sh.at[pl.ds(sid * 640, 640)],
                    out_hbm.at[cid, pl.ds(sid * 640, 640)])


_NBUF = 4
_CH = 64


def _extract_half(pk_all, r, half, sring, dring, b):
    mask = jnp.full((16,), 0xFFFF, jnp.int32)
    for k in range(4):
        v = pk_all[r, pl.ds(half * _CH + k * 16, 16)]
        sring[b, pl.ds(k * 16, 16)] = jnp.bitwise_and(v, mask)
        dring[b, pl.ds(k * 16, 16)] = jnp.right_shift(v, 16)


@functools.partial(
    pl.kernel,
    out_type=jax.ShapeDtypeStruct((NC, ACC_ROWS, D), jnp.float32),
    mesh=_mesh,
    scratch_types=(
        [
            pltpu.VMEM((ROWS_PER_TILE, 128), jnp.int32),
            pltpu.VMEM((_NBUF, _CH), jnp.int32),
            pltpu.VMEM((_NBUF, _CH), jnp.int32),
        ]
        + [pltpu.VMEM((_CH, D), jnp.float32)] * _NBUF
        + [
            pltpu.VMEM((16, D), jnp.float32),
            pltpu.VMEM_SHARED((ACC_ROWS, D), jnp.float32),
        ]
        + [pltpu.SemaphoreType.DMA] * (2 * _NBUF)
    ),
)
def _sc_aggregate(h_hbm, eidx_hbm, out_hbm, pk_all, sring, dring, *rest):
    rows = rest[:_NBUF]
    z_v = rest[_NBUF]
    acc_sh = rest[_NBUF + 1]
    gsem = rest[_NBUF + 2:_NBUF + 2 + _NBUF]
    ssem = rest[_NBUF + 2 + _NBUF:]

    cid = lax.axis_index("c")
    sid = lax.axis_index("s")
    wid = _worker_id()
    last = wid == NW - 1
    nchunks = jnp.where(last, 40, 2 * ROWS_PER_TILE)

    @pl.when(last)
    def _():
        pltpu.sync_copy(eidx_hbm.at[pl.ds((NW - 1) * ROWS_PER_TILE, 20)],
                        pk_all.at[pl.ds(0, 20)])

    @pl.when(jnp.logical_not(last))
    def _():
        pltpu.sync_copy(
            eidx_hbm.at[pl.ds(wid * ROWS_PER_TILE, ROWS_PER_TILE)], pk_all)

    for b in range(_NBUF):
        _extract_half(pk_all, b >> 1, b & 1, sring, dring, b)
        pltpu.async_copy(h_hbm.at[sring.at[b]], rows[b], gsem[b])

    zero16 = jnp.zeros((16,), jnp.float32)
    for r in range(16):
        for c in range(8):
            z_v[r, pl.ds(c * 16, 16)] = zero16

    def zcp(t, carry):
        pltpu.sync_copy(z_v, acc_sh.at[pl.ds(sid * 640 + t * 16, 16)])
        return carry

    lax.fori_loop(0, 40, zcp, 0)
    plsc.subcore_barrier()

    def blk(g, carry):
        for b in range(_NBUF):
            c = g * _NBUF + b
            pltpu.make_async_copy(h_hbm.at[sring.at[b]], rows[b],
                                  gsem[b]).wait()
            pltpu.async_copy(rows[b], acc_sh.at[dring.at[b]], ssem[b],
                             add=True)

            @pl.when(c + _NBUF < nchunks)
            def _():
                pltpu.make_async_copy(rows[b], acc_sh.at[dring.at[b]],
                                      ssem[b]).wait()
                _extract_half(pk_all, 2 * (g + 1) + (b >> 1), b & 1,
                              sring, dring, b)
                pltpu.async_copy(h_hbm.at[sring.at[b]], rows[b], gsem[b])
        return carry

    lax.fori_loop(0, nchunks // _NBUF, blk, 0)
    for b in range(_NBUF):
        pltpu.make_async_copy(rows[b], acc_sh.at[dring.at[b]],
                              ssem[b]).wait()
    plsc.subcore_barrier()
    pltpu.sync_copy(acc_sh.at[pl.ds(sid * 640, 640)],
                    out_hbm.at[cid, pl.ds(sid * 640, 640)])


_GRID = 10
_BR = N_NODES // _GRID


def _dis_of(degp_ref):
    deg = degp_ref[:, 0] + degp_ref[:, 1] + 1.0
    return lax.rsqrt(deg)


def _tc1_body(x_ref, w_ref, degp_ref, o_ref):
    dis = _dis_of(degp_ref)
    h = jnp.dot(x_ref[...], w_ref[...], preferred_element_type=jnp.float32)
    o_ref[...] = h * dis[:, None]


def _tc2_body(agg_ref, hp_ref, degp_ref, b_ref, w_ref, o_ref):
    dis = _dis_of(degp_ref)
    t = (agg_ref[0] + agg_ref[1] + hp_ref[...]) * dis[:, None] + b_ref[...]
    t = jnp.maximum(t, 0.0)
    h = jnp.dot(t, w_ref[...], preferred_element_type=jnp.float32)
    o_ref[...] = h * dis[:, None]


def _tc3_body(agg_ref, hp_ref, degp_ref, b_ref, o_ref):
    dis = _dis_of(degp_ref)
    o_ref[...] = ((agg_ref[0] + agg_ref[1] + hp_ref[...]) * dis[:, None]
                  + b_ref[...])


def _pack_body(e_ref, o_ref):
    o_ref[...] = jnp.bitwise_or(e_ref[0], jnp.left_shift(e_ref[1], 16))


_tc_pack = pl.pallas_call(
    _pack_body,
    in_specs=[pl.BlockSpec((2, IDX_ROWS, 128), lambda: (0, 0, 0))],
    out_specs=pl.BlockSpec((IDX_ROWS, 128), lambda: (0, 0)),
    out_shape=jax.ShapeDtypeStruct((IDX_ROWS, 128), jnp.int32),
)


_ROWS_SPEC = pl.BlockSpec((_BR, D), lambda i: (i, 0))
_W_SPEC = pl.BlockSpec((D, D), lambda i: (0, 0))
_DEG_SPEC = pl.BlockSpec((_BR, NC), lambda i: (i, 0))
_AGG_SPEC = pl.BlockSpec((NC, _BR, D), lambda i: (0, i, 0))
_B_SPEC = pl.BlockSpec((1, D), lambda i: (0, 0))

_tc1 = pl.pallas_call(
    _tc1_body,
    grid=(_GRID,),
    in_specs=[_ROWS_SPEC, _W_SPEC, _DEG_SPEC],
    out_specs=_ROWS_SPEC,
    out_shape=jax.ShapeDtypeStruct((N_NODES, D), jnp.float32),
)

_tc2 = pl.pallas_call(
    _tc2_body,
    grid=(_GRID,),
    in_specs=[_AGG_SPEC, _ROWS_SPEC, _DEG_SPEC, _B_SPEC, _W_SPEC],
    out_specs=_ROWS_SPEC,
    out_shape=jax.ShapeDtypeStruct((N_NODES, D), jnp.float32),
)

_tc3 = pl.pallas_call(
    _tc3_body,
    grid=(_GRID,),
    in_specs=[_AGG_SPEC, _ROWS_SPEC, _DEG_SPEC, _B_SPEC],
    out_specs=_ROWS_SPEC,
    out_shape=jax.ShapeDtypeStruct((N_NODES, D), jnp.float32),
)


def kernel(x, edge_index, W1, b1, W2, b2):
    ei = edge_index.astype(jnp.int32).reshape(2, IDX_ROWS, 128)

    b1r = b1.reshape(1, D)
    b2r = b2.reshape(1, D)

    degp = _sc_degree(ei).T
    eidx = _tc_pack(ei)
    h1 = _tc1(x, W1, degp)
    agg1 = _sc_aggregate(h1, eidx)
    h2 = _tc2(agg1, h1, degp, b1r, W2)
    agg2 = _sc_aggregate(h2, eidx)
    return _tc3(agg2, h2, degp, b2r)


# speedup vs baseline: 1.1635x; 1.0018x over previous
import functools

import jax
import jax.numpy as jnp
from jax import lax
from jax.experimental import pallas as pl
from jax.experimental.pallas import tpu as pltpu
from jax.experimental.pallas import tpu_sc as plsc

N_NODES = 10000
N_EDGES = 320000
D = 128

NC = 2
NS = 16
NW = NC * NS

IDX_ROWS = N_EDGES // 128
ROWS_PER_TILE = 80

ACC_ROWS = 10240

_mesh = plsc.VectorSubcoreMesh(core_axis_name="c", subcore_axis_name="s")


def _worker_id():
    return lax.axis_index("c") * NS + lax.axis_index("s")


def _extract_row(pk_all, c, dst_ring=None, b=0, src_ring=None):
    mask = jnp.full((16,), 0xFFFF, jnp.int32)
    for k in range(8):
        v = pk_all[c, pl.ds(k * 16, 16)]
        if src_ring is not None:
            src_ring[b, pl.ds(k * 16, 16)] = jnp.bitwise_and(v, mask)
        if dst_ring is not None:
            dst_ring[b, pl.ds(k * 16, 16)] = jnp.right_shift(v, 16)


@functools.partial(
    pl.kernel,
    out_type=jax.ShapeDtypeStruct((NC, ACC_ROWS), jnp.float32),
    mesh=_mesh,
    scratch_types=[
        pltpu.VMEM((ROWS_PER_TILE, 128), jnp.int32),
        pltpu.VMEM((128,), jnp.float32),
        pltpu.VMEM((640,), jnp.float32),
        pltpu.VMEM_SHARED((ACC_ROWS,), jnp.float32),
        pltpu.SemaphoreType.DMA,
    ],
)
def _sc_degree(ei_hbm, out_hbm, didx_all, ones_v, z_v, acc_sh, sem):
    cid = lax.axis_index("c")
    sid = lax.axis_index("s")
    wid = _worker_id()
    last = wid == NW - 1

    @pl.when(last)
    def _():
        pltpu.sync_copy(ei_hbm.at[1, pl.ds((NW - 1) * ROWS_PER_TILE, 20)],
                        didx_all.at[pl.ds(0, 20)])

    @pl.when(jnp.logical_not(last))
    def _():
        pltpu.sync_copy(
            ei_hbm.at[1, pl.ds(wid * ROWS_PER_TILE, ROWS_PER_TILE)],
            didx_all)

    one16 = jnp.ones((16,), jnp.float32)
    zero16 = jnp.zeros((16,), jnp.float32)
    for j in range(8):
        ones_v[pl.ds(j * 16, 16)] = one16
    for j in range(40):
        z_v[pl.ds(j * 16, 16)] = zero16

    pltpu.sync_copy(z_v, acc_sh.at[pl.ds(sid * 640, 640)])
    plsc.subcore_barrier()

    ngroups = jnp.where(last, 2, ROWS_PER_TILE // 10)

    def blk(g, carry):
        for b in range(10):
            pltpu.async_copy(ones_v, acc_sh.at[didx_all.at[g * 10 + b]],
                             sem, add=True)
        for b in range(10):
            pltpu.make_async_copy(ones_v, acc_sh.at[didx_all.at[0]],
                                  sem).wait()
        return carry

    lax.fori_loop(0, ngroups, blk, 0)
    plsc.subcore_barrier()
    pltpu.sync_copy(acc_sh.at[pl.ds(sid * 640, 640)],
                    out_hbm.at[cid, pl.ds(sid * 640, 640)])


_NBUF = 4
_CH = 64


def _extract_half(pk_all, r, half, sring, dring, b):
    mask = jnp.full((16,), 0xFFFF, jnp.int32)
    for k in range(4):
        v = pk_all[r, pl.ds(half * _CH + k * 16, 16)]
        sring[b, pl.ds(k * 16, 16)] = jnp.bitwise_and(v, mask)
        dring[b, pl.ds(k * 16, 16)] = jnp.right_shift(v, 16)


@functools.partial(
    pl.kernel,
    out_type=jax.ShapeDtypeStruct((NC, ACC_ROWS, D), jnp.float32),
    mesh=_mesh,
    scratch_types=(
        [
            pltpu.VMEM((ROWS_PER_TILE, 128), jnp.int32),
            pltpu.VMEM((_NBUF, _CH), jnp.int32),
            pltpu.VMEM((_NBUF, _CH), jnp.int32),
        ]
        + [pltpu.VMEM((_CH, D), jnp.float32)] * _NBUF
        + [
            pltpu.VMEM((16, D), jnp.float32),
            pltpu.VMEM_SHARED((ACC_ROWS, D), jnp.float32),
        ]
        + [pltpu.SemaphoreType.DMA] * (2 * _NBUF)
    ),
)
def _sc_aggregate(h_hbm, eidx_hbm, out_hbm, pk_all, sring, dring, *rest):
    rows = rest[:_NBUF]
    z_v = rest[_NBUF]
    acc_sh = rest[_NBUF + 1]
    gsem = rest[_NBUF + 2:_NBUF + 2 + _NBUF]
    ssem = rest[_NBUF + 2 + _NBUF:]

    cid = lax.axis_index("c")
    sid = lax.axis_index("s")
    wid = _worker_id()
    last = wid == NW - 1
    nchunks = jnp.where(last, 40, 2 * ROWS_PER_TILE)

    @pl.when(last)
    def _():
        pltpu.sync_copy(eidx_hbm.at[pl.ds((NW - 1) * ROWS_PER_TILE, 20)],
                        pk_all.at[pl.ds(0, 20)])

    @pl.when(jnp.logical_not(last))
    def _():
        pltpu.sync_copy(
            eidx_hbm.at[pl.ds(wid * ROWS_PER_TILE, ROWS_PER_TILE)], pk_all)

    for b in range(_NBUF):
        _extract_half(pk_all, b >> 1, b & 1, sring, dring, b)
        pltpu.async_copy(h_hbm.at[sring.at[b]], rows[b], gsem[b])

    zero16 = jnp.zeros((16,), jnp.float32)
    for r in range(16):
        for c in range(8):
            z_v[r, pl.ds(c * 16, 16)] = zero16

    def zcp(t, carry):
        pltpu.sync_copy(z_v, acc_sh.at[pl.ds(sid * 640 + t * 16, 16)])
        return carry

    lax.fori_loop(0, 40, zcp, 0)
    plsc.subcore_barrier()

    def blk(g, carry):
        for b in range(_NBUF):
            c = g * _NBUF + b
            pltpu.make_async_copy(h_hbm.at[sring.at[b]], rows[b],
                                  gsem[b]).wait()
            pltpu.async_copy(rows[b], acc_sh.at[dring.at[b]], ssem[b],
                             add=True)

            @pl.when(c + _NBUF < nchunks)
            def _():
                pltpu.make_async_copy(rows[b], acc_sh.at[dring.at[b]],
                                      ssem[b]).wait()
                _extract_half(pk_all, 2 * (g + 1) + (b >> 1), b & 1,
                              sring, dring, b)
                pltpu.async_copy(h_hbm.at[sring.at[b]], rows[b], gsem[b])
        return carry

    lax.fori_loop(0, nchunks // _NBUF, blk, 0)
    for b in range(_NBUF):
        pltpu.make_async_copy(rows[b], acc_sh.at[dring.at[b]],
                              ssem[b]).wait()
    plsc.subcore_barrier()
    pltpu.sync_copy(acc_sh.at[pl.ds(sid * 640, 640)],
                    out_hbm.at[cid, pl.ds(sid * 640, 640)])


_GRID = 10
_BR = N_NODES // _GRID


def _dis_of(degp_ref):
    deg = degp_ref[:, 0] + degp_ref[:, 1] + 1.0
    return lax.rsqrt(deg)


def _tc1_body(x_ref, w_ref, degp_ref, o_ref):
    dis = _dis_of(degp_ref)
    h = jnp.dot(x_ref[...], w_ref[...], preferred_element_type=jnp.float32)
    o_ref[...] = h * dis[:, None]


def _tc2_body(agg_ref, hp_ref, degp_ref, b_ref, w_ref, o_ref):
    dis = _dis_of(degp_ref)
    t = (agg_ref[0] + agg_ref[1] + hp_ref[...]) * dis[:, None] + b_ref[...]
    t = jnp.maximum(t, 0.0)
    h = jnp.dot(t, w_ref[...], preferred_element_type=jnp.float32)
    o_ref[...] = h * dis[:, None]


def _tc3_body(agg_ref, hp_ref, degp_ref, b_ref, o_ref):
    dis = _dis_of(degp_ref)
    o_ref[...] = ((agg_ref[0] + agg_ref[1] + hp_ref[...]) * dis[:, None]
                  + b_ref[...])


def _pack_body(e_ref, o_ref):
    o_ref[...] = jnp.bitwise_or(e_ref[0], jnp.left_shift(e_ref[1], 16))


_tc_pack = pl.pallas_call(
    _pack_body,
    in_specs=[pl.BlockSpec((2, IDX_ROWS, 128), lambda: (0, 0, 0))],
    out_specs=pl.BlockSpec((IDX_ROWS, 128), lambda: (0, 0)),
    out_shape=jax.ShapeDtypeStruct((IDX_ROWS, 128), jnp.int32),
)


_ROWS_SPEC = pl.BlockSpec((_BR, D), lambda i: (i, 0))
_W_SPEC = pl.BlockSpec((D, D), lambda i: (0, 0))
_DEG_SPEC = pl.BlockSpec((_BR, NC), lambda i: (i, 0))
_AGG_SPEC = pl.BlockSpec((NC, _BR, D), lambda i: (0, i, 0))
_B_SPEC = pl.BlockSpec((1, D), lambda i: (0, 0))

_tc1 = pl.pallas_call(
    _tc1_body,
    grid=(_GRID,),
    in_specs=[_ROWS_SPEC, _W_SPEC, _DEG_SPEC],
    out_specs=_ROWS_SPEC,
    out_shape=jax.ShapeDtypeStruct((N_NODES, D), jnp.float32),
)

_tc2 = pl.pallas_call(
    _tc2_body,
    grid=(_GRID,),
    in_specs=[_AGG_SPEC, _ROWS_SPEC, _DEG_SPEC, _B_SPEC, _W_SPEC],
    out_specs=_ROWS_SPEC,
    out_shape=jax.ShapeDtypeStruct((N_NODES, D), jnp.float32),
)

_tc3 = pl.pallas_call(
    _tc3_body,
    grid=(_GRID,),
    in_specs=[_AGG_SPEC, _ROWS_SPEC, _DEG_SPEC, _B_SPEC],
    out_specs=_ROWS_SPEC,
    out_shape=jax.ShapeDtypeStruct((N_NODES, D), jnp.float32),
)


def kernel(x, edge_index, W1, b1, W2, b2):
    ei = edge_index.astype(jnp.int32).reshape(2, IDX_ROWS, 128)

    b1r = b1.reshape(1, D)
    b2r = b2.reshape(1, D)

    degp = _sc_degree(ei).T
    eidx = _tc_pack(ei)
    h1 = _tc1(x, W1, degp)
    agg1 = _sc_aggregate(h1, eidx)
    h2 = _tc2(agg1, h1, degp, b1r, W2)
    agg2 = _sc_aggregate(h2, eidx)
    return _tc3(agg2, h2, degp, b2r)


# speedup vs baseline: 1.1643x; 1.0007x over previous
import functools

import jax
import jax.numpy as jnp
from jax import lax
from jax.experimental import pallas as pl
from jax.experimental.pallas import tpu as pltpu
from jax.experimental.pallas import tpu_sc as plsc

N_NODES = 10000
N_EDGES = 320000
D = 128

NC = 2
NS = 16
NW = NC * NS

IDX_ROWS = N_EDGES // 128
ROWS_PER_TILE = 80

ACC_ROWS = 10240

_mesh = plsc.VectorSubcoreMesh(core_axis_name="c", subcore_axis_name="s")


def _worker_id():
    return lax.axis_index("c") * NS + lax.axis_index("s")


def _extract_row(pk_all, c, dst_ring=None, b=0, src_ring=None):
    mask = jnp.full((16,), 0xFFFF, jnp.int32)
    for k in range(8):
        v = pk_all[c, pl.ds(k * 16, 16)]
        if src_ring is not None:
            src_ring[b, pl.ds(k * 16, 16)] = jnp.bitwise_and(v, mask)
        if dst_ring is not None:
            dst_ring[b, pl.ds(k * 16, 16)] = jnp.right_shift(v, 16)


@functools.partial(
    pl.kernel,
    out_type=jax.ShapeDtypeStruct((NC, ACC_ROWS), jnp.float32),
    mesh=_mesh,
    scratch_types=[
        pltpu.VMEM((ROWS_PER_TILE, 128), jnp.int32),
        pltpu.VMEM((128,), jnp.float32),
        pltpu.VMEM((640,), jnp.float32),
        pltpu.VMEM_SHARED((ACC_ROWS,), jnp.float32),
        pltpu.SemaphoreType.DMA,
    ],
)
def _sc_degree(ei_hbm, out_hbm, didx_all, ones_v, z_v, acc_sh, sem):
    cid = lax.axis_index("c")
    sid = lax.axis_index("s")
    wid = _worker_id()
    last = wid == NW - 1

    @pl.when(last)
    def _():
        pltpu.sync_copy(ei_hbm.at[1, pl.ds((NW - 1) * ROWS_PER_TILE, 20)],
                        didx_all.at[pl.ds(0, 20)])

    @pl.when(jnp.logical_not(last))
    def _():
        pltpu.sync_copy(
            ei_hbm.at[1, pl.ds(wid * ROWS_PER_TILE, ROWS_PER_TILE)],
            didx_all)

    one16 = jnp.ones((16,), jnp.float32)
    zero16 = jnp.zeros((16,), jnp.float32)
    for j in range(8):
        ones_v[pl.ds(j * 16, 16)] = one16
    for j in range(40):
        z_v[pl.ds(j * 16, 16)] = zero16

    pltpu.sync_copy(z_v, acc_sh.at[pl.ds(sid * 640, 640)])
    plsc.subcore_barrier()

    ngroups = jnp.where(last, 2, ROWS_PER_TILE // 10)

    def blk(g, carry):
        for b in range(10):
            pltpu.async_copy(ones_v, acc_sh.at[didx_all.at[g * 10 + b]],
                             sem, add=True)
        for b in range(10):
            pltpu.make_async_copy(ones_v, acc_sh.at[didx_all.at[0]],
                                  sem).wait()
        return carry

    lax.fori_loop(0, ngroups, blk, 0)
    plsc.subcore_barrier()
    pltpu.sync_copy(acc_sh.at[pl.ds(sid * 640, 640)],
                    out_hbm.at[cid, pl.ds(sid * 640, 640)])


_NBUF = 8
_CH = 32


def _extract_half(pk_all, r, quarter, sring, dring, b):
    mask = jnp.full((16,), 0xFFFF, jnp.int32)
    for k in range(2):
        v = pk_all[r, pl.ds(quarter * _CH + k * 16, 16)]
        sring[b, pl.ds(k * 16, 16)] = jnp.bitwise_and(v, mask)
        dring[b, pl.ds(k * 16, 16)] = jnp.right_shift(v, 16)


@functools.partial(
    pl.kernel,
    out_type=jax.ShapeDtypeStruct((NC, ACC_ROWS, D), jnp.float32),
    mesh=_mesh,
    scratch_types=(
        [
            pltpu.VMEM((ROWS_PER_TILE, 128), jnp.int32),
            pltpu.VMEM((_NBUF, _CH), jnp.int32),
            pltpu.VMEM((_NBUF, _CH), jnp.int32),
        ]
        + [pltpu.VMEM((_CH, D), jnp.float32)] * _NBUF
        + [
            pltpu.VMEM((16, D), jnp.float32),
            pltpu.VMEM_SHARED((ACC_ROWS, D), jnp.float32),
        ]
        + [pltpu.SemaphoreType.DMA] * (2 * _NBUF)
    ),
)
def _sc_aggregate(h_hbm, eidx_hbm, out_hbm, pk_all, sring, dring, *rest):
    rows = rest[:_NBUF]
    z_v = rest[_NBUF]
    acc_sh = rest[_NBUF + 1]
    gsem = rest[_NBUF + 2:_NBUF + 2 + _NBUF]
    ssem = rest[_NBUF + 2 + _NBUF:]

    cid = lax.axis_index("c")
    sid = lax.axis_index("s")
    wid = _worker_id()
    last = wid == NW - 1
    nchunks = jnp.where(last, 80, 4 * ROWS_PER_TILE)

    @pl.when(last)
    def _():
        pltpu.sync_copy(eidx_hbm.at[pl.ds((NW - 1) * ROWS_PER_TILE, 20)],
                        pk_all.at[pl.ds(0, 20)])

    @pl.when(jnp.logical_not(last))
    def _():
        pltpu.sync_copy(
            eidx_hbm.at[pl.ds(wid * ROWS_PER_TILE, ROWS_PER_TILE)], pk_all)

    for b in range(_NBUF):
        _extract_half(pk_all, b >> 2, b & 3, sring, dring, b)
        pltpu.async_copy(h_hbm.at[sring.at[b]], rows[b], gsem[b])

    zero16 = jnp.zeros((16,), jnp.float32)
    for r in range(16):
        for c in range(8):
            z_v[r, pl.ds(c * 16, 16)] = zero16

    def zcp(t, carry):
        pltpu.sync_copy(z_v, acc_sh.at[pl.ds(sid * 640 + t * 16, 16)])
        return carry

    lax.fori_loop(0, 40, zcp, 0)
    plsc.subcore_barrier()

    def blk(g, carry):
        for b in range(_NBUF):
            c = g * _NBUF + b
            pltpu.make_async_copy(h_hbm.at[sring.at[b]], rows[b],
                                  gsem[b]).wait()
            pltpu.async_copy(rows[b], acc_sh.at[dring.at[b]], ssem[b],
                             add=True)

            @pl.when(c + _NBUF < nchunks)
            def _():
                pltpu.make_async_copy(rows[b], acc_sh.at[dring.at[b]],
                                      ssem[b]).wait()
                _extract_half(pk_all, 2 * (g + 1) + (b >> 2), b & 3,
                              sring, dring, b)
                pltpu.async_copy(h_hbm.at[sring.at[b]], rows[b], gsem[b])
        return carry

    lax.fori_loop(0, nchunks // _NBUF, blk, 0)
    for b in range(_NBUF):
        pltpu.make_async_copy(rows[b], acc_sh.at[dring.at[b]],
                              ssem[b]).wait()
    plsc.subcore_barrier()
    pltpu.sync_copy(acc_sh.at[pl.ds(sid * 640, 640)],
                    out_hbm.at[cid, pl.ds(sid * 640, 640)])


_GRID = 10
_BR = N_NODES // _GRID


def _dis_of(degp_ref):
    deg = degp_ref[:, 0] + degp_ref[:, 1] + 1.0
    return lax.rsqrt(deg)


def _tc1_body(x_ref, w_ref, degp_ref, o_ref):
    dis = _dis_of(degp_ref)
    h = jnp.dot(x_ref[...], w_ref[...], preferred_element_type=jnp.float32)
    o_ref[...] = h * dis[:, None]


def _tc2_body(agg_ref, hp_ref, degp_ref, b_ref, w_ref, o_ref):
    dis = _dis_of(degp_ref)
    t = (agg_ref[0] + agg_ref[1] + hp_ref[...]) * dis[:, None] + b_ref[...]
    t = jnp.maximum(t, 0.0)
    h = jnp.dot(t, w_ref[...], preferred_element_type=jnp.float32)
    o_ref[...] = h * dis[:, None]


def _tc3_body(agg_ref, hp_ref, degp_ref, b_ref, o_ref):
    dis = _dis_of(degp_ref)
    o_ref[...] = ((agg_ref[0] + agg_ref[1] + hp_ref[...]) * dis[:, None]
                  + b_ref[...])


def _pack_body(e_ref, o_ref):
    o_ref[...] = jnp.bitwise_or(e_ref[0], jnp.left_shift(e_ref[1], 16))


_tc_pack = pl.pallas_call(
    _pack_body,
    in_specs=[pl.BlockSpec((2, IDX_ROWS, 128), lambda: (0, 0, 0))],
    out_specs=pl.BlockSpec((IDX_ROWS, 128), lambda: (0, 0)),
    out_shape=jax.ShapeDtypeStruct((IDX_ROWS, 128), jnp.int32),
)


_ROWS_SPEC = pl.BlockSpec((_BR, D), lambda i: (i, 0))
_W_SPEC = pl.BlockSpec((D, D), lambda i: (0, 0))
_DEG_SPEC = pl.BlockSpec((_BR, NC), lambda i: (i, 0))
_AGG_SPEC = pl.BlockSpec((NC, _BR, D), lambda i: (0, i, 0))
_B_SPEC = pl.BlockSpec((1, D), lambda i: (0, 0))

_tc1 = pl.pallas_call(
    _tc1_body,
    grid=(_GRID,),
    in_specs=[_ROWS_SPEC, _W_SPEC, _DEG_SPEC],
    out_specs=_ROWS_SPEC,
    out_shape=jax.ShapeDtypeStruct((N_NODES, D), jnp.float32),
)

_tc2 = pl.pallas_call(
    _tc2_body,
    grid=(_GRID,),
    in_specs=[_AGG_SPEC, _ROWS_SPEC, _DEG_SPEC, _B_SPEC, _W_SPEC],
    out_specs=_ROWS_SPEC,
    out_shape=jax.ShapeDtypeStruct((N_NODES, D), jnp.float32),
)

_tc3 = pl.pallas_call(
    _tc3_body,
    grid=(_GRID,),
    in_specs=[_AGG_SPEC, _ROWS_SPEC, _DEG_SPEC, _B_SPEC],
    out_specs=_ROWS_SPEC,
    out_shape=jax.ShapeDtypeStruct((N_NODES, D), jnp.float32),
)


def kernel(x, edge_index, W1, b1, W2, b2):
    ei = edge_index.astype(jnp.int32).reshape(2, IDX_ROWS, 128)

    b1r = b1.reshape(1, D)
    b2r = b2.reshape(1, D)

    degp = _sc_degree(ei).T
    eidx = _tc_pack(ei)
    h1 = _tc1(x, W1, degp)
    agg1 = _sc_aggregate(h1, eidx)
    h2 = _tc2(agg1, h1, degp, b1r, W2)
    agg2 = _sc_aggregate(h2, eidx)
    return _tc3(agg2, h2, degp, b2r)
